# baseline (speedup 1.0000x reference)
import functools

import jax
import jax.numpy as jnp
from jax import lax
from jax.experimental import pallas as pl
from jax.experimental.pallas import tpu as pltpu
from jax.experimental.pallas import tpu_sc as plsc

VOCAB = 1000000
EMBED_DIM = 32
BATCH = 16384
HIST = 50

_HP = 64
_NC, _NS = 2, 16
_NW = _NC * _NS
_BPW = BATCH // _NW
_BBLK = 16
_NBLK = _BPW // _BBLK
_NBUF = 2


def _embed_kernel(w_hbm, x_hbm, out_hbm, idx_v, rows_v, isem, gsem, osem):
  wid = lax.axis_index("s") * _NC + lax.axis_index("c")
  b_base = wid * _BPW

  def fire_idx(m, b):
    pltpu.async_copy(
        x_hbm.at[pl.ds(b_base + m * _BBLK, _BBLK)], idx_v.at[b], isem.at[b]
    )

  def wait_idx(b):
    pltpu.make_async_copy(
        x_hbm.at[pl.ds(0, _BBLK)], idx_v.at[b], isem.at[b]
    ).wait()

  def fire_gathers(b):
    for i in range(_BBLK):
      pltpu.async_copy(
          w_hbm.at[idx_v.at[b, i]], rows_v.at[b, i], gsem.at[b]
      )

  def drain_gathers(b):
    for i in range(_BBLK):
      pltpu.make_async_copy(
          w_hbm.at[idx_v.at[b, i]], rows_v.at[b, i], gsem.at[b]
      ).wait()

  def fire_out(m, b):
    pltpu.async_copy(
        rows_v.at[b],
        out_hbm.at[pl.ds(b_base + m * _BBLK, _BBLK)],
        osem.at[b],
    )

  def wait_out(b):
    pltpu.make_async_copy(
        rows_v.at[b], out_hbm.at[pl.ds(0, _BBLK)], osem.at[b]
    ).wait()

  for g in range(_NBUF):
    fire_idx(g, g)
  wait_idx(0)
  fire_gathers(0)

  @pl.loop(0, _NBLK, step=_NBUF)
  def _ring(m0):
    for b0 in range(_NBUF):
      m = m0 + b0
      nb = (b0 + 1) % _NBUF
      @pl.when(m + 1 < _NBLK)
      def _():
        @pl.when(m > 0)
        def _():
          wait_out(nb)
        wait_idx(nb)
        fire_gathers(nb)
      drain_gathers(b0)
      fire_out(m, b0)
      @pl.when(m + _NBUF < _NBLK)
      def _():
        fire_idx(m + _NBUF, b0)

  for b in range(_NBUF):
    wait_out(b)


@jax.jit
def _embed(weight, xp):
  mesh = plsc.VectorSubcoreMesh(core_axis_name="c", subcore_axis_name="s")
  run = pl.kernel(
      _embed_kernel,
      out_type=jax.ShapeDtypeStruct((BATCH, _HP, EMBED_DIM), jnp.float32),
      mesh=mesh,
      scratch_types=[
          pltpu.VMEM((_NBUF, _BBLK, _HP), jnp.int32),
          pltpu.VMEM((_NBUF, _BBLK, _HP, EMBED_DIM), jnp.float32),
          pltpu.SemaphoreType.DMA((_NBUF,)),
          pltpu.SemaphoreType.DMA((_NBUF,)),
          pltpu.SemaphoreType.DMA((_NBUF,)),
      ],
      compiler_params=pltpu.CompilerParams(use_tc_tiling_on_sc=False),
  )
  return run(weight, xp)


def kernel(x, weight):
  xp = jnp.pad(x.astype(jnp.int32), ((0, 0), (0, _HP - HIST)))
  outp = _embed(weight, xp)
  return outp[:, :HIST, :]


# speedup vs baseline: 2.2576x; 2.2576x over previous
import functools

import jax
import jax.numpy as jnp
from jax import lax
from jax.experimental import pallas as pl
from jax.experimental.pallas import tpu as pltpu
from jax.experimental.pallas import tpu_sc as plsc

VOCAB = 1000000
EMBED_DIM = 32
BATCH = 16384
HIST = 50

_B = BATCH * HIST
_NC, _NS = 2, 16
_NW = _NC * _NS
_PER_W = _B // _NW
_CHUNK = 512
_NBUF = 5
_MACRO = _PER_W // _CHUNK


def _embed_kernel(table_hbm, idx_hbm, out_hbm, idx_v, rows_v, gsem, osem):
  wid = lax.axis_index("s") * _NC + lax.axis_index("c")
  pltpu.sync_copy(idx_hbm.at[pl.ds(wid * _PER_W, _PER_W)], idx_v)
  row_base = wid * _PER_W

  def fire_gather(m, b):
    pltpu.async_copy(
        table_hbm.at[idx_v.at[pl.ds(m * _CHUNK, _CHUNK)]],
        rows_v.at[b],
        gsem.at[b],
    )

  def drain_gather(b):
    pltpu.make_async_copy(
        table_hbm.at[idx_v.at[pl.ds(0, _CHUNK)]],
        rows_v.at[b],
        gsem.at[b],
    ).wait()

  def fire_out(m, b):
    pltpu.async_copy(
        rows_v.at[b],
        out_hbm.at[pl.ds(row_base + m * _CHUNK, _CHUNK)],
        osem.at[b],
    )

  def wait_out(b):
    pltpu.make_async_copy(
        rows_v.at[b],
        out_hbm.at[pl.ds(0, _CHUNK)],
        osem.at[b],
    ).wait()

  for g in range(_NBUF - 1):
    fire_gather(g, g)

  @pl.loop(0, _MACRO, step=_NBUF)
  def _ring(m0):
    for b0 in range(_NBUF):
      m = m0 + b0
      nb = (b0 + _NBUF - 1) % _NBUF
      @pl.when(m + _NBUF - 1 < _MACRO)
      def _():
        @pl.when(m > 0)
        def _():
          wait_out(nb)
        fire_gather(m + _NBUF - 1, nb)
      drain_gather(b0)
      fire_out(m, b0)

  for b in range(_NBUF):
    wait_out(b)


@jax.jit
def _embed(weight, idx):
  mesh = plsc.VectorSubcoreMesh(core_axis_name="c", subcore_axis_name="s")
  run = pl.kernel(
      _embed_kernel,
      out_type=jax.ShapeDtypeStruct((_B, EMBED_DIM), jnp.float32),
      mesh=mesh,
      scratch_types=[
          pltpu.VMEM((_PER_W,), jnp.int32),
          pltpu.VMEM((_NBUF, _CHUNK, EMBED_DIM), jnp.float32),
          pltpu.SemaphoreType.DMA((_NBUF,)),
          pltpu.SemaphoreType.DMA((_NBUF,)),
      ],
      compiler_params=pltpu.CompilerParams(use_tc_tiling_on_sc=False),
  )
  return run(weight, idx)


def kernel(x, weight):
  idx = x.reshape(-1).astype(jnp.int32)
  flat = _embed(weight, idx)
  return flat.reshape(x.shape + (EMBED_DIM,))
